# baseline (speedup 1.0000x reference)
import functools

import jax
import jax.numpy as jnp
from jax import lax
from jax.experimental import pallas as pl
from jax.experimental.pallas import tpu as pltpu
from jax.experimental.pallas import tpu_sc as plsc

_NC, _NS = 2, 16
_NW = _NC * _NS
_L = 16


def _bias_gather_body(bw, u_hbm, i_hbm, ub_hbm, ib_hbm,
                      bu_out, bi_out,
                      uidx_v, iidx_v, bu_v, bi_v, sem):
    wid = lax.axis_index("s") * _NC + lax.axis_index("c")
    base = wid * bw
    pltpu.sync_copy(u_hbm.at[pl.ds(base, bw)], uidx_v)
    pltpu.sync_copy(i_hbm.at[pl.ds(base, bw)], iidx_v)
    c1 = pltpu.async_copy(ub_hbm.at[uidx_v], bu_v, sem)
    c2 = pltpu.async_copy(ib_hbm.at[iidx_v], bi_v, sem)
    c1.wait()
    pltpu.sync_copy(bu_v, bu_out.at[pl.ds(base, bw)])
    c2.wait()
    pltpu.sync_copy(bi_v, bi_out.at[pl.ds(base, bw)])


def _sc_bias_gather(u, i, user_bias, item_bias):
    B = u.shape[0]
    bw = B // _NW
    mesh = plsc.VectorSubcoreMesh(core_axis_name="c", subcore_axis_name="s",
                                  num_cores=_NC, num_subcores=_NS)
    f32 = jnp.float32
    k = pl.kernel(
        functools.partial(_bias_gather_body, bw),
        out_type=(
            jax.ShapeDtypeStruct((B,), f32),
            jax.ShapeDtypeStruct((B,), f32),
        ),
        mesh=mesh,
        scratch_types=[
            pltpu.VMEM((bw,), jnp.int32),
            pltpu.VMEM((bw,), jnp.int32),
            pltpu.VMEM((bw,), f32),
            pltpu.VMEM((bw,), f32),
            pltpu.SemaphoreType.DMA,
        ],
        compiler_params=pltpu.CompilerParams(use_tc_tiling_on_sc=False),
    )
    return k(u, i, user_bias, item_bias)


def _vec_gather_body(bw, idx_hbm, tab_hbm, rows_out,
                     idx_v, q_v, rows_v, sem):
    wid = lax.axis_index("s") * _NC + lax.axis_index("c")
    base = wid * bw
    pltpu.sync_copy(idx_hbm.at[pl.ds(base, bw)], idx_v)
    for k in range(bw // _L):
        sl = pl.ds(k * _L, _L)
        q_v[sl] = lax.shift_right_logical(idx_v[sl], 2)
    pltpu.async_copy(tab_hbm.at[q_v], rows_v, sem).wait()
    pltpu.sync_copy(rows_v, rows_out.at[pl.ds(base, bw)])


def _sc_vec_gather(idx, tab128):
    B = idx.shape[0]
    bw = B // _NW
    mesh = plsc.VectorSubcoreMesh(core_axis_name="c", subcore_axis_name="s",
                                  num_cores=_NC, num_subcores=_NS)
    k = pl.kernel(
        functools.partial(_vec_gather_body, bw),
        out_type=jax.ShapeDtypeStruct((B, 128), jnp.float32),
        mesh=mesh,
        scratch_types=[
            pltpu.VMEM((bw,), jnp.int32),
            pltpu.VMEM((bw,), jnp.int32),
            pltpu.VMEM((bw, 128), jnp.float32),
            pltpu.SemaphoreType.DMA,
        ],
        compiler_params=pltpu.CompilerParams(use_tc_tiling_on_sc=True),
    )
    return k(idx, tab128)


def _dense_body(gu_ref, gi_ref, u_ref, i_ref, bu_ref, bi_ref,
                w1t_ref, b1_ref, w2t_ref, b2_ref, gb_ref, out_ref):
    gu = gu_ref[...]
    gi = gi_ref[...]
    mu = (u_ref[...] & 3)[:, None]
    mi = (i_ref[...] & 3)[:, None]
    D = 32
    vu = jnp.where(mu == 0, gu[:, 0:D], 0.0)
    vi = jnp.where(mi == 0, gi[:, 0:D], 0.0)
    for j in range(1, 4):
        vu = vu + jnp.where(mu == j, gu[:, j * D:(j + 1) * D], 0.0)
        vi = vi + jnp.where(mi == j, gi[:, j * D:(j + 1) * D], 0.0)
    h1 = jnp.dot(vu, w1t_ref[...], preferred_element_type=jnp.float32)
    h1 = h1 + b1_ref[...]
    h2 = jnp.dot(vi, w2t_ref[...], preferred_element_type=jnp.float32)
    h2 = h2 + b2_ref[...]
    s = jnp.sum(h1 * h2, axis=1)
    out_ref[...] = s + bu_ref[...] + bi_ref[...] + gb_ref[0, 0]


def _tc_dense(gu, gi, u, i, bu, bi, W1, b1, W2, b2, glob_bias):
    B = gu.shape[0]
    nb = 8
    bb = B // nb
    vec = lambda: pl.BlockSpec((bb,), lambda b: (b,))
    full = lambda shp: pl.BlockSpec(shp, lambda b: tuple(0 for _ in shp))
    return pl.pallas_call(
        _dense_body,
        grid=(nb,),
        in_specs=[
            pl.BlockSpec((bb, 128), lambda b: (b, 0)),
            pl.BlockSpec((bb, 128), lambda b: (b, 0)),
            vec(), vec(), vec(), vec(),
            full((32, 32)), full((1, 32)), full((32, 32)), full((1, 32)),
            full((1, 1)),
        ],
        out_specs=vec(),
        out_shape=jax.ShapeDtypeStruct((B,), jnp.float32),
    )(gu, gi, u, i, bu, bi, W1.T, b1.reshape(1, -1), W2.T,
      b2.reshape(1, -1), glob_bias)


def kernel(u, i, glob_bias, user_bias, user_vec, item_bias, item_vec,
           W1, b1, W2, b2):
    V, D = user_vec.shape
    uv128 = user_vec.reshape(V * D // 128, 128)
    iv128 = item_vec.reshape(V * D // 128, 128)
    bu, bi = _sc_bias_gather(u, i, user_bias, item_bias)
    gu = _sc_vec_gather(u, uv128)
    gi = _sc_vec_gather(i, iv128)
    return _tc_dense(gu, gi, u, i, bu, bi, W1, b1, W2, b2, glob_bias)


# speedup vs baseline: 1.0896x; 1.0896x over previous
import functools

import jax
import jax.numpy as jnp
from jax import lax
from jax.experimental import pallas as pl
from jax.experimental.pallas import tpu as pltpu
from jax.experimental.pallas import tpu_sc as plsc

_NC, _NS = 2, 16
_NW = _NC * _NS
_L = 16


def _bias_gather_body(bw, u_hbm, i_hbm, ub_hbm, ib_hbm,
                      bu_out, bi_out,
                      uidx_v, iidx_v, bu_v, bi_v, sem):
    wid = lax.axis_index("s") * _NC + lax.axis_index("c")
    base = wid * bw
    pltpu.sync_copy(u_hbm.at[pl.ds(base, bw)], uidx_v)
    pltpu.sync_copy(i_hbm.at[pl.ds(base, bw)], iidx_v)
    c1 = pltpu.async_copy(ub_hbm.at[uidx_v], bu_v, sem)
    c2 = pltpu.async_copy(ib_hbm.at[iidx_v], bi_v, sem)
    c1.wait()
    pltpu.sync_copy(bu_v, bu_out.at[pl.ds(base, bw)])
    c2.wait()
    pltpu.sync_copy(bi_v, bi_out.at[pl.ds(base, bw)])


def _sc_bias_gather(u, i, user_bias, item_bias):
    B = u.shape[0]
    bw = B // _NW
    mesh = plsc.VectorSubcoreMesh(core_axis_name="c", subcore_axis_name="s",
                                  num_cores=_NC, num_subcores=_NS)
    f32 = jnp.float32
    k = pl.kernel(
        functools.partial(_bias_gather_body, bw),
        out_type=(
            jax.ShapeDtypeStruct((B,), f32),
            jax.ShapeDtypeStruct((B,), f32),
        ),
        mesh=mesh,
        scratch_types=[
            pltpu.VMEM((bw,), jnp.int32),
            pltpu.VMEM((bw,), jnp.int32),
            pltpu.VMEM((bw,), f32),
            pltpu.VMEM((bw,), f32),
            pltpu.SemaphoreType.DMA,
        ],
        compiler_params=pltpu.CompilerParams(use_tc_tiling_on_sc=False),
    )
    return k(u, i, user_bias, item_bias)


def _vec_gather_body(bw, idx_hbm, tab_hbm, rows_out,
                     idx_v, q_v, rows_v, sem):
    wid = lax.axis_index("s") * _NC + lax.axis_index("c")
    base = wid * bw
    pltpu.sync_copy(idx_hbm.at[pl.ds(base, bw)], idx_v)
    for k in range(bw // _L):
        sl = pl.ds(k * _L, _L)
        q_v[sl] = lax.shift_right_logical(idx_v[sl], 2)
    pltpu.async_copy(tab_hbm.at[q_v], rows_v, sem).wait()
    pltpu.sync_copy(rows_v, rows_out.at[pl.ds(base, bw)])


def _sc_vec_gather(idx, tab128):
    B = idx.shape[0]
    bw = B // _NW
    mesh = plsc.VectorSubcoreMesh(core_axis_name="c", subcore_axis_name="s",
                                  num_cores=_NC, num_subcores=_NS)
    k = pl.kernel(
        functools.partial(_vec_gather_body, bw),
        out_type=jax.ShapeDtypeStruct((B, 128), jnp.float32),
        mesh=mesh,
        scratch_types=[
            pltpu.VMEM((bw,), jnp.int32),
            pltpu.VMEM((bw,), jnp.int32),
            pltpu.VMEM((bw, 128), jnp.float32),
            pltpu.SemaphoreType.DMA,
        ],
        compiler_params=pltpu.CompilerParams(use_tc_tiling_on_sc=True),
    )
    return k(idx, tab128)


def _relayout_body(C, in_ref, out_ref):
    t = in_ref[...].T
    t3 = t.reshape(C // 4, 4, 32)
    out_ref[...] = jnp.concatenate([t3[:, j, :] for j in range(4)], axis=1)


def _tc_relayout(uvT):
    V = uvT.shape[1]
    C = 4096
    return pl.pallas_call(
        functools.partial(_relayout_body, C),
        grid=((V + C - 1) // C,),
        in_specs=[pl.BlockSpec((32, C), lambda b: (0, b))],
        out_specs=pl.BlockSpec((C // 4, 128), lambda b: (b, 0)),
        out_shape=jax.ShapeDtypeStruct((V * 32 // 128, 128), jnp.float32),
    )(uvT)


def _dense_body(gu_ref, gi_ref, u_ref, i_ref, bu_ref, bi_ref,
                w1t_ref, b1_ref, w2t_ref, b2_ref, gb_ref, out_ref):
    gu = gu_ref[...]
    gi = gi_ref[...]
    mu = (u_ref[...] & 3)[:, None]
    mi = (i_ref[...] & 3)[:, None]
    D = 32
    vu = jnp.where(mu == 0, gu[:, 0:D], 0.0)
    vi = jnp.where(mi == 0, gi[:, 0:D], 0.0)
    for j in range(1, 4):
        vu = vu + jnp.where(mu == j, gu[:, j * D:(j + 1) * D], 0.0)
        vi = vi + jnp.where(mi == j, gi[:, j * D:(j + 1) * D], 0.0)
    h1 = jnp.dot(vu, w1t_ref[...], preferred_element_type=jnp.float32)
    h1 = h1 + b1_ref[...]
    h2 = jnp.dot(vi, w2t_ref[...], preferred_element_type=jnp.float32)
    h2 = h2 + b2_ref[...]
    s = jnp.sum(h1 * h2, axis=1)
    out_ref[...] = s + bu_ref[...] + bi_ref[...] + gb_ref[0, 0]


def _tc_dense(gu, gi, u, i, bu, bi, W1, b1, W2, b2, glob_bias):
    B = gu.shape[0]
    nb = 8
    bb = B // nb
    vec = lambda: pl.BlockSpec((bb,), lambda b: (b,))
    full = lambda shp: pl.BlockSpec(shp, lambda b: tuple(0 for _ in shp))
    return pl.pallas_call(
        _dense_body,
        grid=(nb,),
        in_specs=[
            pl.BlockSpec((bb, 128), lambda b: (b, 0)),
            pl.BlockSpec((bb, 128), lambda b: (b, 0)),
            vec(), vec(), vec(), vec(),
            full((32, 32)), full((1, 32)), full((32, 32)), full((1, 32)),
            full((1, 1)),
        ],
        out_specs=vec(),
        out_shape=jax.ShapeDtypeStruct((B,), jnp.float32),
    )(gu, gi, u, i, bu, bi, W1.T, b1.reshape(1, -1), W2.T,
      b2.reshape(1, -1), glob_bias)


def kernel(u, i, glob_bias, user_bias, user_vec, item_bias, item_vec,
           W1, b1, W2, b2):
    uv128 = _tc_relayout(user_vec.T)
    iv128 = _tc_relayout(item_vec.T)
    bu, bi = _sc_bias_gather(u, i, user_bias, item_bias)
    gu = _sc_vec_gather(u, uv128)
    gi = _sc_vec_gather(i, iv128)
    return _tc_dense(gu, gi, u, i, bu, bi, W1, b1, W2, b2, glob_bias)


# speedup vs baseline: 2.0603x; 1.8908x over previous
import functools

import jax
import jax.numpy as jnp
from jax import lax
from jax.experimental import pallas as pl
from jax.experimental.pallas import tpu as pltpu
from jax.experimental.pallas import tpu_sc as plsc

_NC, _NS = 2, 16
_NW = _NC * _NS
_L = 16


def _bias_gather_body(bw, u_hbm, i_hbm, ub_hbm, ib_hbm,
                      bu_out, bi_out,
                      uidx_v, iidx_v, bu_v, bi_v, sem):
    wid = lax.axis_index("s") * _NC + lax.axis_index("c")
    base = wid * bw
    pltpu.sync_copy(u_hbm.at[pl.ds(base, bw)], uidx_v)
    pltpu.sync_copy(i_hbm.at[pl.ds(base, bw)], iidx_v)
    c1 = pltpu.async_copy(ub_hbm.at[uidx_v], bu_v, sem)
    c2 = pltpu.async_copy(ib_hbm.at[iidx_v], bi_v, sem)
    c1.wait()
    pltpu.sync_copy(bu_v, bu_out.at[pl.ds(base, bw)])
    c2.wait()
    pltpu.sync_copy(bi_v, bi_out.at[pl.ds(base, bw)])


def _sc_bias_gather(u, i, user_bias, item_bias):
    B = u.shape[0]
    bw = B // _NW
    mesh = plsc.VectorSubcoreMesh(core_axis_name="c", subcore_axis_name="s",
                                  num_cores=_NC, num_subcores=_NS)
    f32 = jnp.float32
    k = pl.kernel(
        functools.partial(_bias_gather_body, bw),
        out_type=(
            jax.ShapeDtypeStruct((B,), f32),
            jax.ShapeDtypeStruct((B,), f32),
        ),
        mesh=mesh,
        scratch_types=[
            pltpu.VMEM((bw,), jnp.int32),
            pltpu.VMEM((bw,), jnp.int32),
            pltpu.VMEM((bw,), f32),
            pltpu.VMEM((bw,), f32),
            pltpu.SemaphoreType.DMA,
        ],
        compiler_params=pltpu.CompilerParams(use_tc_tiling_on_sc=False),
    )
    return k(u, i, user_bias, item_bias)


def _vec_gather_body(bw, idx_hbm, tab_hbm, rows_out,
                     idx_v, q_v, rows_v, sem):
    wid = lax.axis_index("s") * _NC + lax.axis_index("c")
    base = wid * bw
    pltpu.sync_copy(idx_hbm.at[pl.ds(base, bw)], idx_v)
    for k in range(bw // _L):
        sl = pl.ds(k * _L, _L)
        u = idx_v[sl]
        q_v[sl] = lax.shift_left(lax.shift_right_logical(u, 12), 10) | (u & 1023)
    pltpu.async_copy(tab_hbm.at[q_v], rows_v, sem).wait()
    pltpu.sync_copy(rows_v, rows_out.at[pl.ds(base, bw)])


def _sc_vec_gather(idx, tab128):
    B = idx.shape[0]
    bw = B // _NW
    mesh = plsc.VectorSubcoreMesh(core_axis_name="c", subcore_axis_name="s",
                                  num_cores=_NC, num_subcores=_NS)
    k = pl.kernel(
        functools.partial(_vec_gather_body, bw),
        out_type=jax.ShapeDtypeStruct((B, 128), jnp.float32),
        mesh=mesh,
        scratch_types=[
            pltpu.VMEM((bw,), jnp.int32),
            pltpu.VMEM((bw,), jnp.int32),
            pltpu.VMEM((bw, 128), jnp.float32),
            pltpu.SemaphoreType.DMA,
        ],
        compiler_params=pltpu.CompilerParams(use_tc_tiling_on_sc=True),
    )
    return k(idx, tab128)


def _relayout_body(C, in_ref, out_ref):
    x = in_ref[...]
    q = C // 4
    t = jnp.concatenate([x[:, c * q:(c + 1) * q] for c in range(4)],
                        axis=0)
    out_ref[...] = t.T


def _tc_relayout(uvT):
    V = uvT.shape[1]
    C = 4096
    G = (V + C - 1) // C
    return pl.pallas_call(
        functools.partial(_relayout_body, C),
        grid=(G,),
        in_specs=[pl.BlockSpec((32, C), lambda b: (0, b))],
        out_specs=pl.BlockSpec((C // 4, 128), lambda b: (b, 0)),
        out_shape=jax.ShapeDtypeStruct((G * (C // 4), 128), jnp.float32),
    )(uvT)


def _dense_body(gu_ref, gi_ref, u_ref, i_ref, bu_ref, bi_ref,
                w1t_ref, b1_ref, w2t_ref, b2_ref, gb_ref, out_ref):
    gu = gu_ref[...]
    gi = gi_ref[...]
    mu = (lax.shift_right_logical(u_ref[...], 10) & 3)[:, None]
    mi = (lax.shift_right_logical(i_ref[...], 10) & 3)[:, None]
    D = 32
    vu = jnp.where(mu == 0, gu[:, 0:D], 0.0)
    vi = jnp.where(mi == 0, gi[:, 0:D], 0.0)
    for j in range(1, 4):
        vu = vu + jnp.where(mu == j, gu[:, j * D:(j + 1) * D], 0.0)
        vi = vi + jnp.where(mi == j, gi[:, j * D:(j + 1) * D], 0.0)
    h1 = jnp.dot(vu, w1t_ref[...], preferred_element_type=jnp.float32)
    h1 = h1 + b1_ref[...]
    h2 = jnp.dot(vi, w2t_ref[...], preferred_element_type=jnp.float32)
    h2 = h2 + b2_ref[...]
    s = jnp.sum(h1 * h2, axis=1)
    out_ref[...] = s + bu_ref[...] + bi_ref[...] + gb_ref[0, 0]


def _tc_dense(gu, gi, u, i, bu, bi, W1, b1, W2, b2, glob_bias):
    B = gu.shape[0]
    nb = 8
    bb = B // nb
    vec = lambda: pl.BlockSpec((bb,), lambda b: (b,))
    full = lambda shp: pl.BlockSpec(shp, lambda b: tuple(0 for _ in shp))
    return pl.pallas_call(
        _dense_body,
        grid=(nb,),
        in_specs=[
            pl.BlockSpec((bb, 128), lambda b: (b, 0)),
            pl.BlockSpec((bb, 128), lambda b: (b, 0)),
            vec(), vec(), vec(), vec(),
            full((32, 32)), full((1, 32)), full((32, 32)), full((1, 32)),
            full((1, 1)),
        ],
        out_specs=vec(),
        out_shape=jax.ShapeDtypeStruct((B,), jnp.float32),
    )(gu, gi, u, i, bu, bi, W1.T, b1.reshape(1, -1), W2.T,
      b2.reshape(1, -1), glob_bias)


def kernel(u, i, glob_bias, user_bias, user_vec, item_bias, item_vec,
           W1, b1, W2, b2):
    uv128 = _tc_relayout(user_vec.T)
    iv128 = _tc_relayout(item_vec.T)
    bu, bi = _sc_bias_gather(u, i, user_bias, item_bias)
    gu = _sc_vec_gather(u, uv128)
    gi = _sc_vec_gather(i, iv128)
    return _tc_dense(gu, gi, u, i, bu, bi, W1, b1, W2, b2, glob_bias)


# speedup vs baseline: 3.5131x; 1.7052x over previous
import functools

import jax
import jax.numpy as jnp
from jax import lax
from jax.experimental import pallas as pl
from jax.experimental.pallas import tpu as pltpu
from jax.experimental.pallas import tpu_sc as plsc

_NC, _NS = 2, 16
_NW = _NC * _NS
_L = 16


def _bias_gather_body(bw, u_hbm, i_hbm, ub_hbm, ib_hbm,
                      bu_out, bi_out,
                      uidx_v, iidx_v, bu_v, bi_v, sem):
    wid = lax.axis_index("s") * _NC + lax.axis_index("c")
    base = wid * bw
    pltpu.sync_copy(u_hbm.at[pl.ds(base, bw)], uidx_v)
    pltpu.sync_copy(i_hbm.at[pl.ds(base, bw)], iidx_v)
    c1 = pltpu.async_copy(ub_hbm.at[uidx_v], bu_v, sem)
    c2 = pltpu.async_copy(ib_hbm.at[iidx_v], bi_v, sem)
    c1.wait()
    pltpu.sync_copy(bu_v, bu_out.at[pl.ds(base, bw)])
    c2.wait()
    pltpu.sync_copy(bi_v, bi_out.at[pl.ds(base, bw)])


def _sc_bias_gather(u, i, user_bias, item_bias):
    B = u.shape[0]
    bw = B // _NW
    mesh = plsc.VectorSubcoreMesh(core_axis_name="c", subcore_axis_name="s",
                                  num_cores=_NC, num_subcores=_NS)
    f32 = jnp.float32
    k = pl.kernel(
        functools.partial(_bias_gather_body, bw),
        out_type=(
            jax.ShapeDtypeStruct((B,), f32),
            jax.ShapeDtypeStruct((B,), f32),
        ),
        mesh=mesh,
        scratch_types=[
            pltpu.VMEM((bw,), jnp.int32),
            pltpu.VMEM((bw,), jnp.int32),
            pltpu.VMEM((bw,), f32),
            pltpu.VMEM((bw,), f32),
            pltpu.SemaphoreType.DMA,
        ],
        compiler_params=pltpu.CompilerParams(use_tc_tiling_on_sc=False),
    )
    return k(u, i, user_bias, item_bias)


def _vec_gather_body(bw, idx_hbm, tab_hbm, rows_out,
                     idx_v, q_v, rows_v, sem):
    wid = lax.axis_index("s") * _NC + lax.axis_index("c")
    base = wid * bw
    pltpu.sync_copy(idx_hbm.at[pl.ds(base, bw)], idx_v)
    for k in range(bw // _L):
        sl = pl.ds(k * _L, _L)
        u = idx_v[sl]
        q_v[sl] = lax.shift_left(lax.shift_right_logical(u, 14), 12) | (u & 4095)
    pltpu.async_copy(tab_hbm.at[q_v], rows_v, sem).wait()
    pltpu.sync_copy(rows_v, rows_out.at[pl.ds(base, bw)])


def _sc_vec_gather(idx, tab128):
    B = idx.shape[0]
    bw = B // _NW
    mesh = plsc.VectorSubcoreMesh(core_axis_name="c", subcore_axis_name="s",
                                  num_cores=_NC, num_subcores=_NS)
    k = pl.kernel(
        functools.partial(_vec_gather_body, bw),
        out_type=jax.ShapeDtypeStruct((B, 128), jnp.float32),
        mesh=mesh,
        scratch_types=[
            pltpu.VMEM((bw,), jnp.int32),
            pltpu.VMEM((bw,), jnp.int32),
            pltpu.VMEM((bw, 128), jnp.float32),
            pltpu.SemaphoreType.DMA,
        ],
        compiler_params=pltpu.CompilerParams(use_tc_tiling_on_sc=True),
    )
    return k(idx, tab128)


def _relayout_body(C, in_ref, out_ref):
    x = in_ref[...]
    q = C // 4
    t = jnp.concatenate([x[:, c * q:(c + 1) * q] for c in range(4)],
                        axis=0)
    out_ref[...] = t.T


def _tc_relayout(uvT):
    V = uvT.shape[1]
    C = 16384
    G = (V + C - 1) // C
    return pl.pallas_call(
        functools.partial(_relayout_body, C),
        grid=(G,),
        in_specs=[pl.BlockSpec((32, C), lambda b: (0, b))],
        out_specs=pl.BlockSpec((C // 4, 128), lambda b: (b, 0)),
        out_shape=jax.ShapeDtypeStruct((G * (C // 4), 128), jnp.float32),
    )(uvT)


def _dense_body(gu_ref, gi_ref, u_ref, i_ref, bu_ref, bi_ref,
                w1t_ref, b1_ref, w2t_ref, b2_ref, gb_ref, out_ref):
    gu = gu_ref[...]
    gi = gi_ref[...]
    mu = (lax.shift_right_logical(u_ref[...], 12) & 3)[:, None]
    mi = (lax.shift_right_logical(i_ref[...], 12) & 3)[:, None]
    D = 32
    vu = jnp.where(mu == 0, gu[:, 0:D], 0.0)
    vi = jnp.where(mi == 0, gi[:, 0:D], 0.0)
    for j in range(1, 4):
        vu = vu + jnp.where(mu == j, gu[:, j * D:(j + 1) * D], 0.0)
        vi = vi + jnp.where(mi == j, gi[:, j * D:(j + 1) * D], 0.0)
    h1 = jnp.dot(vu, w1t_ref[...], preferred_element_type=jnp.float32)
    h1 = h1 + b1_ref[...]
    h2 = jnp.dot(vi, w2t_ref[...], preferred_element_type=jnp.float32)
    h2 = h2 + b2_ref[...]
    s = jnp.sum(h1 * h2, axis=1)
    out_ref[...] = s + bu_ref[...] + bi_ref[...] + gb_ref[0, 0]


def _tc_dense(gu, gi, u, i, bu, bi, W1, b1, W2, b2, glob_bias):
    B = gu.shape[0]
    nb = 8
    bb = B // nb
    vec = lambda: pl.BlockSpec((bb,), lambda b: (b,))
    full = lambda shp: pl.BlockSpec(shp, lambda b: tuple(0 for _ in shp))
    return pl.pallas_call(
        _dense_body,
        grid=(nb,),
        in_specs=[
            pl.BlockSpec((bb, 128), lambda b: (b, 0)),
            pl.BlockSpec((bb, 128), lambda b: (b, 0)),
            vec(), vec(), vec(), vec(),
            full((32, 32)), full((1, 32)), full((32, 32)), full((1, 32)),
            full((1, 1)),
        ],
        out_specs=vec(),
        out_shape=jax.ShapeDtypeStruct((B,), jnp.float32),
    )(gu, gi, u, i, bu, bi, W1.T, b1.reshape(1, -1), W2.T,
      b2.reshape(1, -1), glob_bias)


def kernel(u, i, glob_bias, user_bias, user_vec, item_bias, item_vec,
           W1, b1, W2, b2):
    uv128 = _tc_relayout(user_vec.T)
    iv128 = _tc_relayout(item_vec.T)
    bu, bi = _sc_bias_gather(u, i, user_bias, item_bias)
    gu = _sc_vec_gather(u, uv128)
    gi = _sc_vec_gather(i, iv128)
    return _tc_dense(gu, gi, u, i, bu, bi, W1, b1, W2, b2, glob_bias)


# speedup vs baseline: 3.9228x; 1.1166x over previous
import functools

import jax
import jax.numpy as jnp
from jax import lax
from jax.experimental import pallas as pl
from jax.experimental.pallas import tpu as pltpu
from jax.experimental.pallas import tpu_sc as plsc

_NC, _NS = 2, 16
_RELAYOUT_C = 32768
_NW = _NC * _NS
_L = 16


def _bias_gather_body(bw, u_hbm, i_hbm, ub_hbm, ib_hbm,
                      bu_out, bi_out,
                      uidx_v, iidx_v, bu_v, bi_v, sem):
    wid = lax.axis_index("s") * _NC + lax.axis_index("c")
    base = wid * bw
    pltpu.sync_copy(u_hbm.at[pl.ds(base, bw)], uidx_v)
    pltpu.sync_copy(i_hbm.at[pl.ds(base, bw)], iidx_v)
    c1 = pltpu.async_copy(ub_hbm.at[uidx_v], bu_v, sem)
    c2 = pltpu.async_copy(ib_hbm.at[iidx_v], bi_v, sem)
    c1.wait()
    pltpu.sync_copy(bu_v, bu_out.at[pl.ds(base, bw)])
    c2.wait()
    pltpu.sync_copy(bi_v, bi_out.at[pl.ds(base, bw)])


def _sc_bias_gather(u, i, user_bias, item_bias):
    B = u.shape[0]
    bw = B // _NW
    mesh = plsc.VectorSubcoreMesh(core_axis_name="c", subcore_axis_name="s",
                                  num_cores=_NC, num_subcores=_NS)
    f32 = jnp.float32
    k = pl.kernel(
        functools.partial(_bias_gather_body, bw),
        out_type=(
            jax.ShapeDtypeStruct((B,), f32),
            jax.ShapeDtypeStruct((B,), f32),
        ),
        mesh=mesh,
        scratch_types=[
            pltpu.VMEM((bw,), jnp.int32),
            pltpu.VMEM((bw,), jnp.int32),
            pltpu.VMEM((bw,), f32),
            pltpu.VMEM((bw,), f32),
            pltpu.SemaphoreType.DMA,
        ],
        compiler_params=pltpu.CompilerParams(use_tc_tiling_on_sc=False),
    )
    return k(u, i, user_bias, item_bias)


def _vec_gather_body(bw, sh_hi, sh_lo, idx_hbm, tab_hbm, rows_out,
                     idx_v, q_v, rows_v, sem):
    wid = lax.axis_index("s") * _NC + lax.axis_index("c")
    base = wid * bw
    pltpu.sync_copy(idx_hbm.at[pl.ds(base, bw)], idx_v)
    for k in range(bw // _L):
        sl = pl.ds(k * _L, _L)
        u = idx_v[sl]
        q_v[sl] = (lax.shift_left(lax.shift_right_logical(u, sh_hi), sh_lo)
                   | (u & ((1 << sh_lo) - 1)))
    pltpu.async_copy(tab_hbm.at[q_v], rows_v, sem).wait()
    pltpu.sync_copy(rows_v, rows_out.at[pl.ds(base, bw)])


def _sc_vec_gather(idx, tab128, sh_hi, sh_lo):
    B = idx.shape[0]
    bw = B // _NW
    mesh = plsc.VectorSubcoreMesh(core_axis_name="c", subcore_axis_name="s",
                                  num_cores=_NC, num_subcores=_NS)
    k = pl.kernel(
        functools.partial(_vec_gather_body, bw, sh_hi, sh_lo),
        out_type=jax.ShapeDtypeStruct((B, 128), jnp.float32),
        mesh=mesh,
        scratch_types=[
            pltpu.VMEM((bw,), jnp.int32),
            pltpu.VMEM((bw,), jnp.int32),
            pltpu.VMEM((bw, 128), jnp.float32),
            pltpu.SemaphoreType.DMA,
        ],
        compiler_params=pltpu.CompilerParams(use_tc_tiling_on_sc=True),
    )
    return k(idx, tab128)


def _relayout_body(C, in_ref, out_ref):
    x = in_ref[...]
    q = C // 4
    t = jnp.concatenate([x[:, c * q:(c + 1) * q] for c in range(4)],
                        axis=0)
    out_ref[...] = t.T


def _tc_relayout(uvT):
    V = uvT.shape[1]
    C = _RELAYOUT_C
    G = (V + C - 1) // C
    return pl.pallas_call(
        functools.partial(_relayout_body, C),
        grid=(G,),
        in_specs=[pl.BlockSpec((32, C), lambda b: (0, b))],
        out_specs=pl.BlockSpec((C // 4, 128), lambda b: (b, 0)),
        out_shape=jax.ShapeDtypeStruct((G * (C // 4), 128), jnp.float32),
    )(uvT)


def _dense_body(sh_lo, gu_ref, gi_ref, u_ref, i_ref, bu_ref, bi_ref,
                w1t_ref, b1_ref, w2t_ref, b2_ref, gb_ref, out_ref):
    gu = gu_ref[...]
    gi = gi_ref[...]
    mu = (lax.shift_right_logical(u_ref[...], sh_lo) & 3)[:, None]
    mi = (lax.shift_right_logical(i_ref[...], sh_lo) & 3)[:, None]
    D = 32
    vu = jnp.where(mu == 0, gu[:, 0:D], 0.0)
    vi = jnp.where(mi == 0, gi[:, 0:D], 0.0)
    for j in range(1, 4):
        vu = vu + jnp.where(mu == j, gu[:, j * D:(j + 1) * D], 0.0)
        vi = vi + jnp.where(mi == j, gi[:, j * D:(j + 1) * D], 0.0)
    h1 = jnp.dot(vu, w1t_ref[...], preferred_element_type=jnp.float32)
    h1 = h1 + b1_ref[...]
    h2 = jnp.dot(vi, w2t_ref[...], preferred_element_type=jnp.float32)
    h2 = h2 + b2_ref[...]
    s = jnp.sum(h1 * h2, axis=1)
    out_ref[...] = s + bu_ref[...] + bi_ref[...] + gb_ref[0, 0]


def _tc_dense(gu, gi, u, i, bu, bi, W1, b1, W2, b2, glob_bias):
    B = gu.shape[0]
    nb = 8
    bb = B // nb
    vec = lambda: pl.BlockSpec((bb,), lambda b: (b,))
    full = lambda shp: pl.BlockSpec(shp, lambda b: tuple(0 for _ in shp))
    sh_lo = _RELAYOUT_C.bit_length() - 3
    return pl.pallas_call(
        functools.partial(_dense_body, sh_lo),
        grid=(nb,),
        in_specs=[
            pl.BlockSpec((bb, 128), lambda b: (b, 0)),
            pl.BlockSpec((bb, 128), lambda b: (b, 0)),
            vec(), vec(), vec(), vec(),
            full((32, 32)), full((1, 32)), full((32, 32)), full((1, 32)),
            full((1, 1)),
        ],
        out_specs=vec(),
        out_shape=jax.ShapeDtypeStruct((B,), jnp.float32),
    )(gu, gi, u, i, bu, bi, W1.T, b1.reshape(1, -1), W2.T,
      b2.reshape(1, -1), glob_bias)


def kernel(u, i, glob_bias, user_bias, user_vec, item_bias, item_vec,
           W1, b1, W2, b2):
    uv128 = _tc_relayout(user_vec.T)
    iv128 = _tc_relayout(item_vec.T)
    sh_hi = _RELAYOUT_C.bit_length() - 1
    sh_lo = sh_hi - 2
    bu, bi = _sc_bias_gather(u, i, user_bias, item_bias)
    gu = _sc_vec_gather(u, uv128, sh_hi, sh_lo)
    gi = _sc_vec_gather(i, iv128, sh_hi, sh_lo)
    return _tc_dense(gu, gi, u, i, bu, bi, W1, b1, W2, b2, glob_bias)


# speedup vs baseline: 3.9660x; 1.0110x over previous
import functools

import jax
import jax.numpy as jnp
from jax import lax
from jax.experimental import pallas as pl
from jax.experimental.pallas import tpu as pltpu
from jax.experimental.pallas import tpu_sc as plsc

_NC, _NS = 2, 16
_RELAYOUT_C = 65536
_NW = _NC * _NS
_L = 16


def _bias_gather_body(bw, u_hbm, i_hbm, ub_hbm, ib_hbm,
                      bu_out, bi_out,
                      uidx_v, iidx_v, bu_v, bi_v, sem):
    wid = lax.axis_index("s") * _NC + lax.axis_index("c")
    base = wid * bw
    pltpu.sync_copy(u_hbm.at[pl.ds(base, bw)], uidx_v)
    pltpu.sync_copy(i_hbm.at[pl.ds(base, bw)], iidx_v)
    c1 = pltpu.async_copy(ub_hbm.at[uidx_v], bu_v, sem)
    c2 = pltpu.async_copy(ib_hbm.at[iidx_v], bi_v, sem)
    c1.wait()
    pltpu.sync_copy(bu_v, bu_out.at[pl.ds(base, bw)])
    c2.wait()
    pltpu.sync_copy(bi_v, bi_out.at[pl.ds(base, bw)])


def _sc_bias_gather(u, i, user_bias, item_bias):
    B = u.shape[0]
    bw = B // _NW
    mesh = plsc.VectorSubcoreMesh(core_axis_name="c", subcore_axis_name="s",
                                  num_cores=_NC, num_subcores=_NS)
    f32 = jnp.float32
    k = pl.kernel(
        functools.partial(_bias_gather_body, bw),
        out_type=(
            jax.ShapeDtypeStruct((B,), f32),
            jax.ShapeDtypeStruct((B,), f32),
        ),
        mesh=mesh,
        scratch_types=[
            pltpu.VMEM((bw,), jnp.int32),
            pltpu.VMEM((bw,), jnp.int32),
            pltpu.VMEM((bw,), f32),
            pltpu.VMEM((bw,), f32),
            pltpu.SemaphoreType.DMA,
        ],
        compiler_params=pltpu.CompilerParams(use_tc_tiling_on_sc=False),
    )
    return k(u, i, user_bias, item_bias)


def _vec_gather_body(bw, sh_hi, sh_lo, idx_hbm, tab_hbm, rows_out,
                     idx_v, q_v, rows_v, sem):
    wid = lax.axis_index("s") * _NC + lax.axis_index("c")
    base = wid * bw
    pltpu.sync_copy(idx_hbm.at[pl.ds(base, bw)], idx_v)
    for k in range(bw // _L):
        sl = pl.ds(k * _L, _L)
        u = idx_v[sl]
        q_v[sl] = (lax.shift_left(lax.shift_right_logical(u, sh_hi), sh_lo)
                   | (u & ((1 << sh_lo) - 1)))
    pltpu.async_copy(tab_hbm.at[q_v], rows_v, sem).wait()
    pltpu.sync_copy(rows_v, rows_out.at[pl.ds(base, bw)])


def _sc_vec_gather(idx, tab128, sh_hi, sh_lo):
    B = idx.shape[0]
    bw = B // _NW
    mesh = plsc.VectorSubcoreMesh(core_axis_name="c", subcore_axis_name="s",
                                  num_cores=_NC, num_subcores=_NS)
    k = pl.kernel(
        functools.partial(_vec_gather_body, bw, sh_hi, sh_lo),
        out_type=jax.ShapeDtypeStruct((B, 128), jnp.float32),
        mesh=mesh,
        scratch_types=[
            pltpu.VMEM((bw,), jnp.int32),
            pltpu.VMEM((bw,), jnp.int32),
            pltpu.VMEM((bw, 128), jnp.float32),
            pltpu.SemaphoreType.DMA,
        ],
        compiler_params=pltpu.CompilerParams(use_tc_tiling_on_sc=True),
    )
    return k(idx, tab128)


def _relayout_body(C, in_ref, out_ref):
    x = in_ref[...]
    q = C // 4
    t = jnp.concatenate([x[:, c * q:(c + 1) * q] for c in range(4)],
                        axis=0)
    out_ref[...] = t.T


def _tc_relayout(uvT):
    V = uvT.shape[1]
    C = _RELAYOUT_C
    G = (V + C - 1) // C
    return pl.pallas_call(
        functools.partial(_relayout_body, C),
        grid=(G,),
        in_specs=[pl.BlockSpec((32, C), lambda b: (0, b))],
        out_specs=pl.BlockSpec((C // 4, 128), lambda b: (b, 0)),
        out_shape=jax.ShapeDtypeStruct((G * (C // 4), 128), jnp.float32),
    )(uvT)


def _dense_body(sh_lo, gu_ref, gi_ref, u_ref, i_ref, bu_ref, bi_ref,
                w1t_ref, b1_ref, w2t_ref, b2_ref, gb_ref, out_ref):
    gu = gu_ref[...]
    gi = gi_ref[...]
    mu = (lax.shift_right_logical(u_ref[...], sh_lo) & 3)[:, None]
    mi = (lax.shift_right_logical(i_ref[...], sh_lo) & 3)[:, None]
    D = 32
    vu = jnp.where(mu == 0, gu[:, 0:D], 0.0)
    vi = jnp.where(mi == 0, gi[:, 0:D], 0.0)
    for j in range(1, 4):
        vu = vu + jnp.where(mu == j, gu[:, j * D:(j + 1) * D], 0.0)
        vi = vi + jnp.where(mi == j, gi[:, j * D:(j + 1) * D], 0.0)
    h1 = jnp.dot(vu, w1t_ref[...], preferred_element_type=jnp.float32)
    h1 = h1 + b1_ref[...]
    h2 = jnp.dot(vi, w2t_ref[...], preferred_element_type=jnp.float32)
    h2 = h2 + b2_ref[...]
    s = jnp.sum(h1 * h2, axis=1)
    out_ref[...] = s + bu_ref[...] + bi_ref[...] + gb_ref[0, 0]


def _tc_dense(gu, gi, u, i, bu, bi, W1, b1, W2, b2, glob_bias):
    B = gu.shape[0]
    nb = 8
    bb = B // nb
    vec = lambda: pl.BlockSpec((bb,), lambda b: (b,))
    full = lambda shp: pl.BlockSpec(shp, lambda b: tuple(0 for _ in shp))
    sh_lo = _RELAYOUT_C.bit_length() - 3
    return pl.pallas_call(
        functools.partial(_dense_body, sh_lo),
        grid=(nb,),
        in_specs=[
            pl.BlockSpec((bb, 128), lambda b: (b, 0)),
            pl.BlockSpec((bb, 128), lambda b: (b, 0)),
            vec(), vec(), vec(), vec(),
            full((32, 32)), full((1, 32)), full((32, 32)), full((1, 32)),
            full((1, 1)),
        ],
        out_specs=vec(),
        out_shape=jax.ShapeDtypeStruct((B,), jnp.float32),
    )(gu, gi, u, i, bu, bi, W1.T, b1.reshape(1, -1), W2.T,
      b2.reshape(1, -1), glob_bias)


def kernel(u, i, glob_bias, user_bias, user_vec, item_bias, item_vec,
           W1, b1, W2, b2):
    uv128 = _tc_relayout(user_vec.T)
    iv128 = _tc_relayout(item_vec.T)
    sh_hi = _RELAYOUT_C.bit_length() - 1
    sh_lo = sh_hi - 2
    bu, bi = _sc_bias_gather(u, i, user_bias, item_bias)
    gu = _sc_vec_gather(u, uv128, sh_hi, sh_lo)
    gi = _sc_vec_gather(i, iv128, sh_hi, sh_lo)
    return _tc_dense(gu, gi, u, i, bu, bi, W1, b1, W2, b2, glob_bias)


# speedup vs baseline: 4.7245x; 1.1913x over previous
import functools

import jax
import jax.numpy as jnp
from jax import lax
from jax.experimental import pallas as pl
from jax.experimental.pallas import tpu as pltpu
from jax.experimental.pallas import tpu_sc as plsc

_NC, _NS = 2, 16
_RELAYOUT_C = 65536
_NW = _NC * _NS
_L = 16


def _bias_gather_body(bw, u_hbm, i_hbm, ub_hbm, ib_hbm,
                      bu_out, bi_out,
                      uidx_v, iidx_v, bu_v, bi_v, sem):
    wid = lax.axis_index("s") * _NC + lax.axis_index("c")
    base = wid * bw
    pltpu.sync_copy(u_hbm.at[pl.ds(base, bw)], uidx_v)
    pltpu.sync_copy(i_hbm.at[pl.ds(base, bw)], iidx_v)
    c1 = pltpu.async_copy(ub_hbm.at[uidx_v], bu_v, sem)
    c2 = pltpu.async_copy(ib_hbm.at[iidx_v], bi_v, sem)
    c1.wait()
    pltpu.sync_copy(bu_v, bu_out.at[pl.ds(base, bw)])
    c2.wait()
    pltpu.sync_copy(bi_v, bi_out.at[pl.ds(base, bw)])


def _sc_bias_gather(u, i, user_bias, item_bias):
    B = u.shape[0]
    bw = B // _NW
    mesh = plsc.VectorSubcoreMesh(core_axis_name="c", subcore_axis_name="s",
                                  num_cores=_NC, num_subcores=_NS)
    f32 = jnp.float32
    k = pl.kernel(
        functools.partial(_bias_gather_body, bw),
        out_type=(
            jax.ShapeDtypeStruct((B,), f32),
            jax.ShapeDtypeStruct((B,), f32),
        ),
        mesh=mesh,
        scratch_types=[
            pltpu.VMEM((bw,), jnp.int32),
            pltpu.VMEM((bw,), jnp.int32),
            pltpu.VMEM((bw,), f32),
            pltpu.VMEM((bw,), f32),
            pltpu.SemaphoreType.DMA,
        ],
        compiler_params=pltpu.CompilerParams(use_tc_tiling_on_sc=False),
    )
    return k(u, i, user_bias, item_bias)


def _vec_gather_body(bw, sh_hi, sh_lo, idx_hbm, tab_hbm, rows_out,
                     idx_v, q_v, rows_v, sem):
    wid = lax.axis_index("s") * _NC + lax.axis_index("c")
    base = wid * bw
    pltpu.sync_copy(idx_hbm.at[pl.ds(base, bw)], idx_v)
    for k in range(bw // _L):
        sl = pl.ds(k * _L, _L)
        u = idx_v[sl]
        q_v[sl] = (lax.shift_left(lax.shift_right_logical(u, sh_hi), sh_lo)
                   | (u & ((1 << sh_lo) - 1)))
    pltpu.async_copy(tab_hbm.at[q_v], rows_v, sem).wait()
    pltpu.sync_copy(rows_v, rows_out.at[pl.ds(base, bw)])


def _sc_vec_gather(idx, tab128, sh_hi, sh_lo):
    B = idx.shape[0]
    bw = B // _NW
    mesh = plsc.VectorSubcoreMesh(core_axis_name="c", subcore_axis_name="s",
                                  num_cores=_NC, num_subcores=_NS)
    k = pl.kernel(
        functools.partial(_vec_gather_body, bw, sh_hi, sh_lo),
        out_type=jax.ShapeDtypeStruct((B, 128), jnp.float32),
        mesh=mesh,
        scratch_types=[
            pltpu.VMEM((bw,), jnp.int32),
            pltpu.VMEM((bw,), jnp.int32),
            pltpu.VMEM((bw, 128), jnp.float32),
            pltpu.SemaphoreType.DMA,
        ],
        compiler_params=pltpu.CompilerParams(use_tc_tiling_on_sc=True),
    )
    return k(idx, tab128)


def _relayout_body(C, in_ref, out_ref):
    x = in_ref[...]
    q = C // 8
    t = jnp.concatenate([x[:, c * q:(c + 1) * q] for c in range(8)],
                        axis=0)
    lo = lax.bitcast_convert_type(t[0:128, :], jnp.uint32)
    hi = lax.bitcast_convert_type(t[128:256, :], jnp.uint32)
    lo16 = lax.shift_right_logical(lo + jnp.uint32(0x8000), jnp.uint32(16))
    hi16 = (hi + jnp.uint32(0x8000)) & jnp.uint32(0xFFFF0000)
    word = lax.bitcast_convert_type(hi16 | lo16, jnp.float32)
    out_ref[...] = word.T


def _tc_relayout(uvT):
    V = uvT.shape[1]
    C = _RELAYOUT_C
    G = (V + C - 1) // C
    return pl.pallas_call(
        functools.partial(_relayout_body, C),
        grid=(G,),
        in_specs=[pl.BlockSpec((32, C), lambda b: (0, b))],
        out_specs=pl.BlockSpec((C // 8, 128), lambda b: (b, 0)),
        out_shape=jax.ShapeDtypeStruct((G * (C // 8), 128), jnp.float32),
    )(uvT)


def _dense_body(sh_lo, gu_ref, gi_ref, u_ref, i_ref, bu_ref, bi_ref,
                w1t_ref, b1_ref, w2t_ref, b2_ref, gb_ref, out_ref):
    gu = gu_ref[...]
    gi = gi_ref[...]
    mu = (lax.shift_right_logical(u_ref[...], sh_lo) & 3)[:, None]
    mi = (lax.shift_right_logical(i_ref[...], sh_lo) & 3)[:, None]
    hu = (lax.shift_right_logical(u_ref[...], sh_lo + 2) & 1)[:, None]
    hi_ = (lax.shift_right_logical(i_ref[...], sh_lo + 2) & 1)[:, None]
    D = 32
    z = jnp.uint32(0)
    gu32 = lax.bitcast_convert_type(gu, jnp.uint32)
    gi32 = lax.bitcast_convert_type(gi, jnp.uint32)
    wu = jnp.where(mu == 0, gu32[:, 0:D], z)
    wi = jnp.where(mi == 0, gi32[:, 0:D], z)
    for j in range(1, 4):
        wu = wu | jnp.where(mu == j, gu32[:, j * D:(j + 1) * D], z)
        wi = wi | jnp.where(mi == j, gi32[:, j * D:(j + 1) * D], z)
    wu = jnp.where(hu == 1, wu & jnp.uint32(0xFFFF0000),
                   lax.shift_left(wu, jnp.uint32(16)))
    wi = jnp.where(hi_ == 1, wi & jnp.uint32(0xFFFF0000),
                   lax.shift_left(wi, jnp.uint32(16)))
    vu = lax.bitcast_convert_type(wu, jnp.float32)
    vi = lax.bitcast_convert_type(wi, jnp.float32)
    h1 = jnp.dot(vu, w1t_ref[...], preferred_element_type=jnp.float32)
    h1 = h1 + b1_ref[...]
    h2 = jnp.dot(vi, w2t_ref[...], preferred_element_type=jnp.float32)
    h2 = h2 + b2_ref[...]
    s = jnp.sum(h1 * h2, axis=1)
    out_ref[...] = s + bu_ref[...] + bi_ref[...] + gb_ref[0, 0]


def _tc_dense(gu, gi, u, i, bu, bi, W1, b1, W2, b2, glob_bias):
    B = gu.shape[0]
    nb = 8
    bb = B // nb
    vec = lambda: pl.BlockSpec((bb,), lambda b: (b,))
    full = lambda shp: pl.BlockSpec(shp, lambda b: tuple(0 for _ in shp))
    sh_lo = _RELAYOUT_C.bit_length() - 4
    return pl.pallas_call(
        functools.partial(_dense_body, sh_lo),
        grid=(nb,),
        in_specs=[
            pl.BlockSpec((bb, 128), lambda b: (b, 0)),
            pl.BlockSpec((bb, 128), lambda b: (b, 0)),
            vec(), vec(), vec(), vec(),
            full((32, 32)), full((1, 32)), full((32, 32)), full((1, 32)),
            full((1, 1)),
        ],
        out_specs=vec(),
        out_shape=jax.ShapeDtypeStruct((B,), jnp.float32),
    )(gu, gi, u, i, bu, bi, W1.T, b1.reshape(1, -1), W2.T,
      b2.reshape(1, -1), glob_bias)


def kernel(u, i, glob_bias, user_bias, user_vec, item_bias, item_vec,
           W1, b1, W2, b2):
    uv128 = _tc_relayout(user_vec.T)
    iv128 = _tc_relayout(item_vec.T)
    sh_hi = _RELAYOUT_C.bit_length() - 1
    sh_lo = sh_hi - 3
    bu, bi = _sc_bias_gather(u, i, user_bias, item_bias)
    gu = _sc_vec_gather(u, uv128, sh_hi, sh_lo)
    gi = _sc_vec_gather(i, iv128, sh_hi, sh_lo)
    return _tc_dense(gu, gi, u, i, bu, bi, W1, b1, W2, b2, glob_bias)


# speedup vs baseline: 4.7679x; 1.0092x over previous
import functools

import jax
import jax.numpy as jnp
from jax import lax
from jax.experimental import pallas as pl
from jax.experimental.pallas import tpu as pltpu
from jax.experimental.pallas import tpu_sc as plsc

_NC, _NS = 2, 16
_RELAYOUT_C = 131072
_NW = _NC * _NS
_L = 16


def _bias_gather_body(bw, u_hbm, i_hbm, ub_hbm, ib_hbm,
                      bu_out, bi_out,
                      uidx_v, iidx_v, bu_v, bi_v, sem):
    wid = lax.axis_index("s") * _NC + lax.axis_index("c")
    base = wid * bw
    pltpu.sync_copy(u_hbm.at[pl.ds(base, bw)], uidx_v)
    pltpu.sync_copy(i_hbm.at[pl.ds(base, bw)], iidx_v)
    c1 = pltpu.async_copy(ub_hbm.at[uidx_v], bu_v, sem)
    c2 = pltpu.async_copy(ib_hbm.at[iidx_v], bi_v, sem)
    c1.wait()
    pltpu.sync_copy(bu_v, bu_out.at[pl.ds(base, bw)])
    c2.wait()
    pltpu.sync_copy(bi_v, bi_out.at[pl.ds(base, bw)])


def _sc_bias_gather(u, i, user_bias, item_bias):
    B = u.shape[0]
    bw = B // _NW
    mesh = plsc.VectorSubcoreMesh(core_axis_name="c", subcore_axis_name="s",
                                  num_cores=_NC, num_subcores=_NS)
    f32 = jnp.float32
    k = pl.kernel(
        functools.partial(_bias_gather_body, bw),
        out_type=(
            jax.ShapeDtypeStruct((B,), f32),
            jax.ShapeDtypeStruct((B,), f32),
        ),
        mesh=mesh,
        scratch_types=[
            pltpu.VMEM((bw,), jnp.int32),
            pltpu.VMEM((bw,), jnp.int32),
            pltpu.VMEM((bw,), f32),
            pltpu.VMEM((bw,), f32),
            pltpu.SemaphoreType.DMA,
        ],
        compiler_params=pltpu.CompilerParams(use_tc_tiling_on_sc=False),
    )
    return k(u, i, user_bias, item_bias)


def _vec_gather_body(bw, sh_hi, sh_lo, idx_hbm, tab_hbm, rows_out,
                     idx_v, q_v, rows_v, sem):
    wid = lax.axis_index("s") * _NC + lax.axis_index("c")
    base = wid * bw
    pltpu.sync_copy(idx_hbm.at[pl.ds(base, bw)], idx_v)
    for k in range(bw // _L):
        sl = pl.ds(k * _L, _L)
        u = idx_v[sl]
        q_v[sl] = (lax.shift_left(lax.shift_right_logical(u, sh_hi), sh_lo)
                   | (u & ((1 << sh_lo) - 1)))
    pltpu.async_copy(tab_hbm.at[q_v], rows_v, sem).wait()
    pltpu.sync_copy(rows_v, rows_out.at[pl.ds(base, bw)])


def _sc_vec_gather(idx, tab128, sh_hi, sh_lo):
    B = idx.shape[0]
    bw = B // _NW
    mesh = plsc.VectorSubcoreMesh(core_axis_name="c", subcore_axis_name="s",
                                  num_cores=_NC, num_subcores=_NS)
    k = pl.kernel(
        functools.partial(_vec_gather_body, bw, sh_hi, sh_lo),
        out_type=jax.ShapeDtypeStruct((B, 128), jnp.float32),
        mesh=mesh,
        scratch_types=[
            pltpu.VMEM((bw,), jnp.int32),
            pltpu.VMEM((bw,), jnp.int32),
            pltpu.VMEM((bw, 128), jnp.float32),
            pltpu.SemaphoreType.DMA,
        ],
        compiler_params=pltpu.CompilerParams(use_tc_tiling_on_sc=True),
    )
    return k(idx, tab128)


def _relayout_body(C, in_ref, out_ref):
    x = in_ref[...]
    q = C // 8
    t = jnp.concatenate([x[:, c * q:(c + 1) * q] for c in range(8)],
                        axis=0)
    lo = lax.bitcast_convert_type(t[0:128, :], jnp.uint32)
    hi = lax.bitcast_convert_type(t[128:256, :], jnp.uint32)
    lo16 = lax.shift_right_logical(lo + jnp.uint32(0x8000), jnp.uint32(16))
    hi16 = (hi + jnp.uint32(0x8000)) & jnp.uint32(0xFFFF0000)
    word = lax.bitcast_convert_type(hi16 | lo16, jnp.float32)
    out_ref[...] = word.T


def _tc_relayout(uvT):
    V = uvT.shape[1]
    C = _RELAYOUT_C
    G = (V + C - 1) // C
    return pl.pallas_call(
        functools.partial(_relayout_body, C),
        grid=(G,),
        in_specs=[pl.BlockSpec((32, C), lambda b: (0, b))],
        out_specs=pl.BlockSpec((C // 8, 128), lambda b: (b, 0)),
        out_shape=jax.ShapeDtypeStruct((G * (C // 8), 128), jnp.float32),
        compiler_params=pltpu.CompilerParams(
            vmem_limit_bytes=128 * 1024 * 1024),
    )(uvT)


def _dense_body(sh_lo, gu_ref, gi_ref, u_ref, i_ref, bu_ref, bi_ref,
                w1t_ref, b1_ref, w2t_ref, b2_ref, gb_ref, out_ref):
    gu = gu_ref[...]
    gi = gi_ref[...]
    mu = (lax.shift_right_logical(u_ref[...], sh_lo) & 3)[:, None]
    mi = (lax.shift_right_logical(i_ref[...], sh_lo) & 3)[:, None]
    hu = (lax.shift_right_logical(u_ref[...], sh_lo + 2) & 1)[:, None]
    hi_ = (lax.shift_right_logical(i_ref[...], sh_lo + 2) & 1)[:, None]
    D = 32
    z = jnp.uint32(0)
    gu32 = lax.bitcast_convert_type(gu, jnp.uint32)
    gi32 = lax.bitcast_convert_type(gi, jnp.uint32)
    wu = jnp.where(mu == 0, gu32[:, 0:D], z)
    wi = jnp.where(mi == 0, gi32[:, 0:D], z)
    for j in range(1, 4):
        wu = wu | jnp.where(mu == j, gu32[:, j * D:(j + 1) * D], z)
        wi = wi | jnp.where(mi == j, gi32[:, j * D:(j + 1) * D], z)
    wu = jnp.where(hu == 1, wu & jnp.uint32(0xFFFF0000),
                   lax.shift_left(wu, jnp.uint32(16)))
    wi = jnp.where(hi_ == 1, wi & jnp.uint32(0xFFFF0000),
                   lax.shift_left(wi, jnp.uint32(16)))
    vu = lax.bitcast_convert_type(wu, jnp.float32)
    vi = lax.bitcast_convert_type(wi, jnp.float32)
    h1 = jnp.dot(vu, w1t_ref[...], preferred_element_type=jnp.float32)
    h1 = h1 + b1_ref[...]
    h2 = jnp.dot(vi, w2t_ref[...], preferred_element_type=jnp.float32)
    h2 = h2 + b2_ref[...]
    s = jnp.sum(h1 * h2, axis=1)
    out_ref[...] = s + bu_ref[...] + bi_ref[...] + gb_ref[0, 0]


def _tc_dense(gu, gi, u, i, bu, bi, W1, b1, W2, b2, glob_bias):
    B = gu.shape[0]
    nb = 8
    bb = B // nb
    vec = lambda: pl.BlockSpec((bb,), lambda b: (b,))
    full = lambda shp: pl.BlockSpec(shp, lambda b: tuple(0 for _ in shp))
    sh_lo = _RELAYOUT_C.bit_length() - 4
    return pl.pallas_call(
        functools.partial(_dense_body, sh_lo),
        grid=(nb,),
        in_specs=[
            pl.BlockSpec((bb, 128), lambda b: (b, 0)),
            pl.BlockSpec((bb, 128), lambda b: (b, 0)),
            vec(), vec(), vec(), vec(),
            full((32, 32)), full((1, 32)), full((32, 32)), full((1, 32)),
            full((1, 1)),
        ],
        out_specs=vec(),
        out_shape=jax.ShapeDtypeStruct((B,), jnp.float32),
    )(gu, gi, u, i, bu, bi, W1.T, b1.reshape(1, -1), W2.T,
      b2.reshape(1, -1), glob_bias)


def kernel(u, i, glob_bias, user_bias, user_vec, item_bias, item_vec,
           W1, b1, W2, b2):
    uv128 = _tc_relayout(user_vec.T)
    iv128 = _tc_relayout(item_vec.T)
    sh_hi = _RELAYOUT_C.bit_length() - 1
    sh_lo = sh_hi - 3
    bu, bi = _sc_bias_gather(u, i, user_bias, item_bias)
    gu = _sc_vec_gather(u, uv128, sh_hi, sh_lo)
    gi = _sc_vec_gather(i, iv128, sh_hi, sh_lo)
    return _tc_dense(gu, gi, u, i, bu, bi, W1, b1, W2, b2, glob_bias)
